# baseline (speedup 1.0000x reference)
import jax
import jax.numpy as jnp
from jax import lax
from jax.experimental import pallas as pl
from jax.experimental.pallas import tpu as pltpu
from jax.experimental.pallas import tpu_sc as plsc

NC = 2
NS = 16
LANES = 16

B = 4
L = 4096
D = 128
NW = NC * NS
POS_W = L // NW
TOK_W = B * POS_W


def _emb_body(x_hbm, wt_hbm, pos_hbm, out_hbm,
              idx_v, pos_v, rows_v,
              sem_i0, sem_i1, sem_i2, sem_i3, sem_p,
              sem_g, sem_s0, sem_s1, sem_s2, sem_s3):
    wid = lax.axis_index("s") * NC + lax.axis_index("c")
    p0 = wid * POS_W

    isems = (sem_i0, sem_i1, sem_i2, sem_i3)
    ssems = (sem_s0, sem_s1, sem_s2, sem_s3)

    pos_cp = pltpu.async_copy(pos_hbm.at[pl.ds(p0, POS_W)], pos_v, sem_p)
    idx_cps = [
        pltpu.async_copy(x_hbm.at[b, pl.ds(p0, POS_W)],
                         idx_v.at[pl.ds(b * POS_W, POS_W)], isems[b])
        for b in range(B)
    ]
    for cp in idx_cps:
        cp.wait()
    g = pltpu.async_copy(wt_hbm.at[idx_v], rows_v, sem_g)
    pos_cp.wait()
    g.wait()

    stores = []
    for b in range(B):
        base = b * POS_W

        def row(r, rc):
            for j in range(D // LANES):
                sl = pl.ds(j * LANES, LANES)
                rows_v[base + r, sl] = rows_v[base + r, sl] + pos_v[r, sl]
            return rc

        lax.fori_loop(0, POS_W, row, 0)
        stores.append(pltpu.async_copy(
            rows_v.at[pl.ds(base, POS_W)],
            out_hbm.at[b, pl.ds(p0, POS_W)], ssems[b]))

    for st in stores:
        st.wait()


_emb = pl.kernel(
    _emb_body,
    out_type=jax.ShapeDtypeStruct((B, L, D), jnp.float32),
    mesh=plsc.VectorSubcoreMesh(
        core_axis_name="c", subcore_axis_name="s", num_cores=NC, num_subcores=NS
    ),
    scratch_types=[
        pltpu.VMEM((TOK_W,), jnp.int32),
        pltpu.VMEM((POS_W, D), jnp.float32),
        pltpu.VMEM((TOK_W, D), jnp.float32),
    ] + [pltpu.SemaphoreType.DMA] * 10,
)


def kernel(x, word_table, pos_table):
    return _emb(x.astype(jnp.int32), word_table, pos_table)


# speedup vs baseline: 1.0218x; 1.0218x over previous
import jax
import jax.numpy as jnp
from jax import lax
from jax.experimental import pallas as pl
from jax.experimental.pallas import tpu as pltpu
from jax.experimental.pallas import tpu_sc as plsc

NC = 2
NS = 16
LANES = 16

B = 4
L = 4096
D = 128
NW = NC * NS
POS_W = L // NW
TOK_W = B * POS_W
HALF = TOK_W // 2


def _emb_body(x_hbm, wt_hbm, pos_hbm, out_hbm,
              idx_v, pos_v, h0_v, h1_v,
              sem_i0, sem_i1, sem_i2, sem_i3, sem_p,
              sem_g0, sem_g1, sem_s0, sem_s1, sem_s2, sem_s3):
    wid = lax.axis_index("s") * NC + lax.axis_index("c")
    p0 = wid * POS_W

    isems = (sem_i0, sem_i1, sem_i2, sem_i3)
    ssems = (sem_s0, sem_s1, sem_s2, sem_s3)
    half_bufs = (h0_v, h1_v)
    gsems = (sem_g0, sem_g1)

    pos_cp = pltpu.async_copy(pos_hbm.at[pl.ds(p0, POS_W)], pos_v, sem_p)
    idx_cps = [
        pltpu.async_copy(x_hbm.at[b, pl.ds(p0, POS_W)],
                         idx_v.at[pl.ds(b * POS_W, POS_W)], isems[b])
        for b in range(B)
    ]
    gathers = []
    for h in range(2):
        idx_cps[2 * h].wait()
        idx_cps[2 * h + 1].wait()
        gathers.append(pltpu.async_copy(
            wt_hbm.at[idx_v.at[pl.ds(h * HALF, HALF)]], half_bufs[h], gsems[h]))

    pos_cp.wait()

    stores = []
    for h in range(2):
        gathers[h].wait()
        buf = half_bufs[h]
        for k in range(2):
            b = 2 * h + k
            base = k * POS_W

            def row(r, rc):
                for j in range(D // LANES):
                    sl = pl.ds(j * LANES, LANES)
                    buf[base + r, sl] = buf[base + r, sl] + pos_v[r, sl]
                return rc

            lax.fori_loop(0, POS_W, row, 0)
            stores.append(pltpu.async_copy(
                buf.at[pl.ds(base, POS_W)],
                out_hbm.at[b, pl.ds(p0, POS_W)], ssems[b]))

    for st in stores:
        st.wait()


_emb = pl.kernel(
    _emb_body,
    out_type=jax.ShapeDtypeStruct((B, L, D), jnp.float32),
    mesh=plsc.VectorSubcoreMesh(
        core_axis_name="c", subcore_axis_name="s", num_cores=NC, num_subcores=NS
    ),
    scratch_types=[
        pltpu.VMEM((TOK_W,), jnp.int32),
        pltpu.VMEM((POS_W, D), jnp.float32),
        pltpu.VMEM((HALF, D), jnp.float32),
        pltpu.VMEM((HALF, D), jnp.float32),
    ] + [pltpu.SemaphoreType.DMA] * 11,
)


def kernel(x, word_table, pos_table):
    return _emb(x.astype(jnp.int32), word_table, pos_table)
